# Initial kernel scaffold; baseline (speedup 1.0000x reference)
import jax
import jax.numpy as jnp
from jax.experimental import pallas as pl


def kernel(edge_index, emb_weight, T_weight, ugr_weight, sps, W1, W2):
    raise NotImplementedError("write your pallas kernel here")


# speedup vs baseline: 3.5515x; 3.5515x over previous
import functools

import jax
import jax.numpy as jnp
from jax import lax
from jax.experimental import pallas as pl
from jax.experimental.pallas import tpu as pltpu
from jax.experimental.pallas import tpu_sc as plsc

_N = 10000
_D = 128
_E = 320000
_NT = 1000
_NI = 5000
_W = 144
_EPS = 1e-15

_NS = 16
_K = 80
_EPW = _E // _NS
_CH = _EPW // _K
_ACCR = 10240
_RPT = _ACCR // _NS


def _tc1_body(t_ref, sps_ref, out_ref):
    T = t_ref[...]
    rest = T[:, 1:]
    x0 = jnp.sqrt(1.0 + jnp.sum(rest * rest, axis=1, keepdims=True))
    y = rest / (x0 + 1.0)
    y2 = jnp.sum(y * y, axis=1, keepdims=True)
    xk = 2.0 * y / (1.0 + y2)
    g = 1.0 / jnp.sqrt(jnp.clip(1.0 - jnp.sum(xk * xk, axis=1, keepdims=True), _EPS))
    G = jnp.concatenate([g * xk, g], axis=1)
    R = jnp.dot(sps_ref[...], G, preferred_element_type=jnp.float32)
    num = R[:, : _D - 1]
    den = jnp.clip(R[:, _D - 1 :], _EPS)
    mean = num / den
    m2 = jnp.sum(mean * mean, axis=1, keepdims=True)
    kp = mean / (1.0 + jnp.sqrt(jnp.clip(1.0 - m2, _EPS)))
    k2 = jnp.sum(kp * kp, axis=1, keepdims=True)
    dn = jnp.clip(1.0 - k2, _EPS)
    out_ref[...] = jnp.concatenate([(1.0 + k2) / dn, 2.0 * kp / dn], axis=1)


def _tc1(T_weight, sps):
    BM = 1000
    return pl.pallas_call(
        _tc1_body,
        grid=(_NI // BM,),
        in_specs=[
            pl.BlockSpec((_NT, _D), lambda i: (0, 0)),
            pl.BlockSpec((BM, _NT), lambda i: (i, 0)),
        ],
        out_specs=pl.BlockSpec((BM, _D), lambda i: (i, 0)),
        out_shape=jax.ShapeDtypeStruct((_NI, _D), jnp.float32),
    )(T_weight, sps)


def _msg(X, Wm):
    rest = X[:, 1:]
    r2 = jnp.sum(rest * rest, axis=1, keepdims=True)
    x0 = jnp.clip(jnp.sqrt(1.0 + r2), 1.0 + 1e-7)
    dist = jnp.log(x0 + jnp.sqrt((x0 - 1.0) * (x0 + 1.0)))
    rn = jnp.sqrt(jnp.clip(r2, _EPS))
    u = dist * rest / rn
    t = jnp.concatenate([jnp.zeros_like(X[:, :1]), u], axis=1)
    return jnp.dot(t, Wm, preferred_element_type=jnp.float32)


def _tc2_body(x1_ref, x2_ref, w1_ref, w2_ref, m0_ref, m1_ref):
    ones = jnp.ones((x1_ref.shape[0], _W - _D), jnp.float32)
    m0_ref[...] = jnp.concatenate([_msg(x1_ref[...], w1_ref[...]), ones], axis=1)
    m1_ref[...] = jnp.concatenate([_msg(x2_ref[...], w2_ref[...]), ones], axis=1)


def _tc2(X1, X2, W1, W2):
    BM = 1000
    return pl.pallas_call(
        _tc2_body,
        grid=(_N // BM,),
        in_specs=[
            pl.BlockSpec((BM, _D), lambda i: (i, 0)),
            pl.BlockSpec((BM, _D), lambda i: (i, 0)),
            pl.BlockSpec((_D, _D), lambda i: (0, 0)),
            pl.BlockSpec((_D, _D), lambda i: (0, 0)),
        ],
        out_specs=[
            pl.BlockSpec((BM, _W), lambda i: (i, 0)),
            pl.BlockSpec((BM, _W), lambda i: (i, 0)),
        ],
        out_shape=[
            jax.ShapeDtypeStruct((_N, _W), jnp.float32),
            jax.ShapeDtypeStruct((_N, _W), jnp.float32),
        ],
    )(X1, X2, W1, W2)


def _sc_scatter(m01, src0, src1, dst, zrows):
    mesh = plsc.VectorSubcoreMesh(core_axis_name="c", subcore_axis_name="s")

    @functools.partial(
        pl.kernel,
        out_type=(
            jax.ShapeDtypeStruct((_ACCR, _W), jnp.float32),
            jax.ShapeDtypeStruct((_ACCR, _W), jnp.float32),
        ),
        mesh=mesh,
        compiler_params=pltpu.CompilerParams(use_tc_tiling_on_sc=False),
        scratch_types=[
            pltpu.VMEM((_K,), jnp.int32),
            pltpu.VMEM((_K,), jnp.int32),
            pltpu.VMEM((_K, _W), jnp.float32),
            pltpu.VMEM_SHARED((_ACCR, _W), jnp.float32),
            pltpu.SemaphoreType.DMA,
        ],
    )
    def scatter_kernel(m01_hbm, src0_hbm, src1_hbm, dst_hbm, z_hbm,
                       out0_hbm, out1_hbm, srcb, dstb, rowb, acc, sem):
        c = lax.axis_index("c")
        s = lax.axis_index("s")
        rb = s * _RPT
        pltpu.sync_copy(z_hbm, acc.at[pl.ds(rb, _RPT)])
        plsc.subcore_barrier()

        ebase = s * _EPW

        def step(i, carry):
            off = ebase + i * _K

            @pl.when(c == 0)
            def _():
                pltpu.sync_copy(src0_hbm.at[pl.ds(off, _K)], srcb)

            @pl.when(c == 1)
            def _():
                pltpu.sync_copy(src1_hbm.at[pl.ds(off, _K)], srcb)

            pltpu.sync_copy(dst_hbm.at[pl.ds(off, _K)], dstb)
            pltpu.async_copy(m01_hbm.at[srcb], rowb, sem).wait()
            pltpu.sync_copy(rowb, acc.at[dstb], add=True)
            return carry

        lax.fori_loop(0, _CH, step, 0)
        plsc.subcore_barrier()

        @pl.when(c == 0)
        def _():
            pltpu.sync_copy(acc.at[pl.ds(rb, _RPT)], out0_hbm.at[pl.ds(rb, _RPT)])

        @pl.when(c == 1)
        def _():
            pltpu.sync_copy(acc.at[pl.ds(rb, _RPT)], out1_hbm.at[pl.ds(rb, _RPT)])

    return scatter_kernel(m01, src0, src1, dst, zrows)


def _emap(v):
    sp = v[:, 1:]
    n = jnp.sqrt(jnp.clip(jnp.sum(sp * sp, axis=1, keepdims=True), _EPS))
    e = jnp.exp(n)
    ei = 1.0 / e
    x0 = 0.5 * (e + ei)
    rest = 0.5 * (e - ei) * sp / n
    return jnp.concatenate([x0, rest], axis=1)


def _tc3_body(a0_ref, a1_ref, m0_ref, m1_ref, out_ref):
    A0 = a0_ref[...]
    A1 = a1_ref[...]
    M0 = m0_ref[...]
    M1 = m1_ref[...]
    deg = A0[:, _D : _D + 1] + M0[:, _D : _D + 1]
    h1 = _emap((A0[:, :_D] + M0[:, :_D]) / deg)
    h2 = _emap((A1[:, :_D] + M1[:, :_D]) / deg)
    out_ref[...] = jnp.concatenate([h1, h2], axis=1)


def _tc3(A0, A1, M0, M1):
    BM = 1000
    return pl.pallas_call(
        _tc3_body,
        grid=(_N // BM,),
        in_specs=[pl.BlockSpec((BM, _W), lambda i: (i, 0))] * 4,
        out_specs=pl.BlockSpec((BM, 2 * _D), lambda i: (i, 0)),
        out_shape=jax.ShapeDtypeStruct((_N, 2 * _D), jnp.float32),
    )(A0, A1, M0, M1)


def kernel(edge_index, emb_weight, T_weight, ugr_weight, sps, W1, W2):
    x2_items = _tc1(T_weight, sps)
    X2 = jnp.concatenate([ugr_weight, x2_items], axis=0)
    M0, M1 = _tc2(emb_weight, X2, W1, W2)
    m01 = jnp.concatenate([M0, M1], axis=0)
    src = edge_index[0]
    dst = edge_index[1]
    zrows = jnp.zeros((_RPT, _W), jnp.float32)
    A0, A1 = _sc_scatter(m01, src, src + _N, dst, zrows)
    return _tc3(A0[:_N], A1[:_N], M0, M1)


# speedup vs baseline: 7.5001x; 2.1118x over previous
import functools

import jax
import jax.numpy as jnp
from jax import lax
from jax.experimental import pallas as pl
from jax.experimental.pallas import tpu as pltpu
from jax.experimental.pallas import tpu_sc as plsc

_N = 10000
_D = 128
_E = 320000
_NT = 1000
_NI = 5000
_W = 144
_EPS = 1e-15

_NS = 16
_K = 80
_EPW = _E // _NS
_CH = _EPW // _K
_ACCR = 10000
_RPT = _ACCR // _NS
_GC = 25
_NG = _CH // _GC


def _tc1_body(t_ref, sps_ref, out_ref):
    T = t_ref[...]
    rest = T[:, 1:]
    x0 = jnp.sqrt(1.0 + jnp.sum(rest * rest, axis=1, keepdims=True))
    y = rest / (x0 + 1.0)
    y2 = jnp.sum(y * y, axis=1, keepdims=True)
    xk = 2.0 * y / (1.0 + y2)
    g = 1.0 / jnp.sqrt(jnp.clip(1.0 - jnp.sum(xk * xk, axis=1, keepdims=True), _EPS))
    G = jnp.concatenate([g * xk, g], axis=1)
    R = jnp.dot(sps_ref[...], G, preferred_element_type=jnp.float32)
    num = R[:, : _D - 1]
    den = jnp.clip(R[:, _D - 1 :], _EPS)
    mean = num / den
    m2 = jnp.sum(mean * mean, axis=1, keepdims=True)
    kp = mean / (1.0 + jnp.sqrt(jnp.clip(1.0 - m2, _EPS)))
    k2 = jnp.sum(kp * kp, axis=1, keepdims=True)
    dn = jnp.clip(1.0 - k2, _EPS)
    out_ref[...] = jnp.concatenate([(1.0 + k2) / dn, 2.0 * kp / dn], axis=1)


def _tc1(T_weight, sps):
    BM = 1000
    return pl.pallas_call(
        _tc1_body,
        grid=(_NI // BM,),
        in_specs=[
            pl.BlockSpec((_NT, _D), lambda i: (0, 0)),
            pl.BlockSpec((BM, _NT), lambda i: (i, 0)),
        ],
        out_specs=pl.BlockSpec((BM, _D), lambda i: (i, 0)),
        out_shape=jax.ShapeDtypeStruct((_NI, _D), jnp.float32),
    )(T_weight, sps)


def _msg(X, Wm):
    rest = X[:, 1:]
    r2 = jnp.sum(rest * rest, axis=1, keepdims=True)
    x0 = jnp.clip(jnp.sqrt(1.0 + r2), 1.0 + 1e-7)
    dist = jnp.log(x0 + jnp.sqrt((x0 - 1.0) * (x0 + 1.0)))
    rn = jnp.sqrt(jnp.clip(r2, _EPS))
    u = dist * rest / rn
    t = jnp.concatenate([jnp.zeros_like(X[:, :1]), u], axis=1)
    return jnp.dot(t, Wm, preferred_element_type=jnp.float32)


def _tc2_body(x1_ref, x2_ref, w1_ref, w2_ref, m0_ref, m1_ref):
    ones = jnp.ones((x1_ref.shape[0], _W - _D), jnp.float32)
    m0_ref[...] = jnp.concatenate([_msg(x1_ref[...], w1_ref[...]), ones], axis=1)
    m1_ref[...] = jnp.concatenate([_msg(x2_ref[...], w2_ref[...]), ones], axis=1)


def _tc2(X1, X2, W1, W2):
    BM = 1000
    return pl.pallas_call(
        _tc2_body,
        grid=(_N // BM,),
        in_specs=[
            pl.BlockSpec((BM, _D), lambda i: (i, 0)),
            pl.BlockSpec((BM, _D), lambda i: (i, 0)),
            pl.BlockSpec((_D, _D), lambda i: (0, 0)),
            pl.BlockSpec((_D, _D), lambda i: (0, 0)),
        ],
        out_specs=[
            pl.BlockSpec((BM, _W), lambda i: (i, 0)),
            pl.BlockSpec((BM, _W), lambda i: (i, 0)),
        ],
        out_shape=[
            jax.ShapeDtypeStruct((_N, _W), jnp.float32),
            jax.ShapeDtypeStruct((_N, _W), jnp.float32),
        ],
    )(X1, X2, W1, W2)


_NBUF = 3


def _sc_scatter(m01, src0, src1, dst, zrows):
    mesh = plsc.VectorSubcoreMesh(core_axis_name="c", subcore_axis_name="s")

    @functools.partial(
        pl.kernel,
        out_type=(
            jax.ShapeDtypeStruct((_ACCR, _W), jnp.float32),
            jax.ShapeDtypeStruct((_ACCR, _W), jnp.float32),
        ),
        mesh=mesh,
        compiler_params=pltpu.CompilerParams(use_tc_tiling_on_sc=False),
        scratch_types=[
            pltpu.VMEM((_GC, _K), jnp.int32),
            pltpu.VMEM((_GC, _K), jnp.int32),
        ]
        + [pltpu.VMEM((_K, _W), jnp.float32)] * _NBUF
        + [pltpu.VMEM_SHARED((_ACCR, _W), jnp.float32)]
        + [pltpu.SemaphoreType.DMA] * (2 * _NBUF),
    )
    def scatter_kernel(m01_hbm, src0_hbm, src1_hbm, dst_hbm, z_hbm,
                       out0_hbm, out1_hbm, srcg, dstg, *rest):
        rowb = list(rest[:_NBUF])
        acc = rest[_NBUF]
        gsem = list(rest[_NBUF + 1 : 2 * _NBUF + 1])
        ssem = list(rest[2 * _NBUF + 1 :])
        c = lax.axis_index("c")
        s = lax.axis_index("s")
        rb = s * _RPT
        pltpu.sync_copy(z_hbm, acc.at[pl.ds(rb, _RPT)])
        plsc.subcore_barrier()

        rowbase = s * _CH

        def group(g, carry):
            gbase = rowbase + g * _GC

            @pl.when(c == 0)
            def _():
                pltpu.sync_copy(src0_hbm.at[pl.ds(gbase, _GC)], srcg)

            @pl.when(c == 1)
            def _():
                pltpu.sync_copy(src1_hbm.at[pl.ds(gbase, _GC)], srcg)

            pltpu.sync_copy(dst_hbm.at[pl.ds(gbase, _GC)], dstg)

            for b in range(_NBUF - 1):
                pltpu.async_copy(m01_hbm.at[srcg.at[b]], rowb[b], gsem[b])

            for b in range(_GC):
                bb = b % _NBUF
                pltpu.make_async_copy(m01_hbm.at[srcg.at[b]], rowb[bb], gsem[bb]).wait()
                if b > 0:
                    pb = (b - 1) % _NBUF
                    pltpu.make_async_copy(
                        rowb[pb], acc.at[dstg.at[0]], ssem[pb]
                    ).wait()
                if b + _NBUF - 1 < _GC:
                    nb = (b + _NBUF - 1) % _NBUF
                    pltpu.async_copy(m01_hbm.at[srcg.at[b + _NBUF - 1]], rowb[nb], gsem[nb])
                pltpu.async_copy(rowb[bb], acc.at[dstg.at[b]], ssem[bb], add=True)

            lastb = (_GC - 1) % _NBUF
            pltpu.make_async_copy(rowb[lastb], acc.at[dstg.at[0]], ssem[lastb]).wait()
            return carry

        lax.fori_loop(0, _NG, group, 0)
        plsc.subcore_barrier()

        @pl.when(c == 0)
        def _():
            pltpu.sync_copy(acc.at[pl.ds(rb, _RPT)], out0_hbm.at[pl.ds(rb, _RPT)])

        @pl.when(c == 1)
        def _():
            pltpu.sync_copy(acc.at[pl.ds(rb, _RPT)], out1_hbm.at[pl.ds(rb, _RPT)])

    return scatter_kernel(m01, src0, src1, dst, zrows)


def _emap(v):
    sp = v[:, 1:]
    n = jnp.sqrt(jnp.clip(jnp.sum(sp * sp, axis=1, keepdims=True), _EPS))
    e = jnp.exp(n)
    ei = 1.0 / e
    x0 = 0.5 * (e + ei)
    rest = 0.5 * (e - ei) * sp / n
    return jnp.concatenate([x0, rest], axis=1)


def _tc3_body(a0_ref, a1_ref, m0_ref, m1_ref, out_ref):
    A0 = a0_ref[...]
    A1 = a1_ref[...]
    M0 = m0_ref[...]
    M1 = m1_ref[...]
    deg = A0[:, _D : _D + 1] + M0[:, _D : _D + 1]
    h1 = _emap((A0[:, :_D] + M0[:, :_D]) / deg)
    h2 = _emap((A1[:, :_D] + M1[:, :_D]) / deg)
    out_ref[...] = jnp.concatenate([h1, h2], axis=1)


def _tc3(A0, A1, M0, M1):
    BM = 1000
    return pl.pallas_call(
        _tc3_body,
        grid=(_N // BM,),
        in_specs=[pl.BlockSpec((BM, _W), lambda i: (i, 0))] * 4,
        out_specs=pl.BlockSpec((BM, 2 * _D), lambda i: (i, 0)),
        out_shape=jax.ShapeDtypeStruct((_N, 2 * _D), jnp.float32),
    )(A0, A1, M0, M1)


def kernel(edge_index, emb_weight, T_weight, ugr_weight, sps, W1, W2):
    x2_items = _tc1(T_weight, sps)
    X2 = jnp.concatenate([ugr_weight, x2_items], axis=0)
    M0, M1 = _tc2(emb_weight, X2, W1, W2)
    m01 = jnp.concatenate([M0, M1], axis=0)
    src = edge_index[0]
    dst = edge_index[1]
    src2d = src.reshape(_E // _K, _K)
    dst2d = dst.reshape(_E // _K, _K)
    zrows = jnp.zeros((_RPT, _W), jnp.float32)
    A0, A1 = _sc_scatter(m01, src2d, src2d + _N, dst2d, zrows)
    return _tc3(A0[:_N], A1[:_N], M0, M1)


# speedup vs baseline: 7.7863x; 1.0382x over previous
import functools

import jax
import jax.numpy as jnp
from jax import lax
from jax.experimental import pallas as pl
from jax.experimental.pallas import tpu as pltpu
from jax.experimental.pallas import tpu_sc as plsc

_N = 10000
_D = 128
_E = 320000
_NT = 1000
_NI = 5000
_W = 144
_EPS = 1e-15

_NS = 16
_K = 80
_EPW = _E // _NS
_CH = _EPW // _K
_ACCR = 10000
_RPT = _ACCR // _NS
_GC = 10
_NG = _CH // _GC


def _tc1_body(t_ref, sps_ref, out_ref):
    T = t_ref[...]
    rest = T[:, 1:]
    x0 = jnp.sqrt(1.0 + jnp.sum(rest * rest, axis=1, keepdims=True))
    y = rest / (x0 + 1.0)
    y2 = jnp.sum(y * y, axis=1, keepdims=True)
    xk = 2.0 * y / (1.0 + y2)
    g = 1.0 / jnp.sqrt(jnp.clip(1.0 - jnp.sum(xk * xk, axis=1, keepdims=True), _EPS))
    G = jnp.concatenate([g * xk, g], axis=1)
    R = jnp.dot(sps_ref[...], G, preferred_element_type=jnp.float32)
    num = R[:, : _D - 1]
    den = jnp.clip(R[:, _D - 1 :], _EPS)
    mean = num / den
    m2 = jnp.sum(mean * mean, axis=1, keepdims=True)
    kp = mean / (1.0 + jnp.sqrt(jnp.clip(1.0 - m2, _EPS)))
    k2 = jnp.sum(kp * kp, axis=1, keepdims=True)
    dn = jnp.clip(1.0 - k2, _EPS)
    out_ref[...] = jnp.concatenate([(1.0 + k2) / dn, 2.0 * kp / dn], axis=1)


def _tc1(T_weight, sps):
    BM = 1000
    return pl.pallas_call(
        _tc1_body,
        grid=(_NI // BM,),
        in_specs=[
            pl.BlockSpec((_NT, _D), lambda i: (0, 0)),
            pl.BlockSpec((BM, _NT), lambda i: (i, 0)),
        ],
        out_specs=pl.BlockSpec((BM, _D), lambda i: (i, 0)),
        out_shape=jax.ShapeDtypeStruct((_NI, _D), jnp.float32),
    )(T_weight, sps)


def _msg(X, Wm):
    rest = X[:, 1:]
    r2 = jnp.sum(rest * rest, axis=1, keepdims=True)
    x0 = jnp.clip(jnp.sqrt(1.0 + r2), 1.0 + 1e-7)
    dist = jnp.log(x0 + jnp.sqrt((x0 - 1.0) * (x0 + 1.0)))
    rn = jnp.sqrt(jnp.clip(r2, _EPS))
    u = dist * rest / rn
    t = jnp.concatenate([jnp.zeros_like(X[:, :1]), u], axis=1)
    return jnp.dot(t, Wm, preferred_element_type=jnp.float32)


def _tc2_body(x1_ref, x2_ref, w1_ref, w2_ref, m0_ref, m1_ref):
    ones = jnp.ones((x1_ref.shape[0], _W - _D), jnp.float32)
    m0_ref[...] = jnp.concatenate([_msg(x1_ref[...], w1_ref[...]), ones], axis=1)
    m1_ref[...] = jnp.concatenate([_msg(x2_ref[...], w2_ref[...]), ones], axis=1)


def _tc2(X1, X2, W1, W2):
    BM = 1000
    return pl.pallas_call(
        _tc2_body,
        grid=(_N // BM,),
        in_specs=[
            pl.BlockSpec((BM, _D), lambda i: (i, 0)),
            pl.BlockSpec((BM, _D), lambda i: (i, 0)),
            pl.BlockSpec((_D, _D), lambda i: (0, 0)),
            pl.BlockSpec((_D, _D), lambda i: (0, 0)),
        ],
        out_specs=[
            pl.BlockSpec((BM, _W), lambda i: (i, 0)),
            pl.BlockSpec((BM, _W), lambda i: (i, 0)),
        ],
        out_shape=[
            jax.ShapeDtypeStruct((_N, _W), jnp.float32),
            jax.ShapeDtypeStruct((_N, _W), jnp.float32),
        ],
    )(X1, X2, W1, W2)


_NBUF = 3
_TRI = 3 * _GC
_NTRI = (_NG - 1) // 3
_CROWS = _E // _K


def _sc_scatter(m01, srcall, dst, zrows):
    mesh = plsc.VectorSubcoreMesh(core_axis_name="c", subcore_axis_name="s")

    @functools.partial(
        pl.kernel,
        out_type=(
            jax.ShapeDtypeStruct((_ACCR, _W), jnp.float32),
            jax.ShapeDtypeStruct((_ACCR, _W), jnp.float32),
        ),
        mesh=mesh,
        compiler_params=pltpu.CompilerParams(use_tc_tiling_on_sc=False),
        scratch_types=[pltpu.VMEM((_GC, _K), jnp.int32)] * 6
        + [pltpu.VMEM((_K, _W), jnp.float32)] * _NBUF
        + [pltpu.VMEM_SHARED((_ACCR, _W), jnp.float32)]
        + [pltpu.SemaphoreType.DMA] * (3 * _NBUF),
    )
    def scatter_kernel(m01_hbm, src_hbm, dst_hbm, z_hbm,
                       out0_hbm, out1_hbm, *rest):
        srcq = list(rest[0:3])
        dstq = list(rest[3:6])
        rowb = list(rest[6 : 6 + _NBUF])
        acc = rest[6 + _NBUF]
        gsem = list(rest[7 + _NBUF : 7 + 2 * _NBUF])
        ssem = list(rest[7 + 2 * _NBUF : 7 + 3 * _NBUF])
        isem = list(rest[7 + 3 * _NBUF :])
        c = lax.axis_index("c")
        s = lax.axis_index("s")
        rb = s * _RPT
        rowbase = s * _CH
        cbase = c * _CROWS

        def load_idx(grow, q, sem):
            pltpu.async_copy(src_hbm.at[pl.ds(cbase + grow, _GC)], srcq[q], sem)
            pltpu.async_copy(dst_hbm.at[pl.ds(grow, _GC)], dstq[q], sem)

        def wait_idx(grow, q, sem):
            pltpu.make_async_copy(
                src_hbm.at[pl.ds(cbase + grow, _GC)], srcq[q], sem
            ).wait()
            pltpu.make_async_copy(dst_hbm.at[pl.ds(grow, _GC)], dstq[q], sem).wait()

        def process(p, q, jbase, nxt):
            for b in range(_GC):
                bb = (p * _GC + b) % _NBUF
                pltpu.make_async_copy(
                    m01_hbm.at[srcq[q].at[b]], rowb[bb], gsem[bb]
                ).wait()
                pb = (p * _GC + b - 1) % _NBUF
                if p == 0 and b == 0:
                    @pl.when(jbase > 0)
                    def _():
                        pltpu.make_async_copy(
                            rowb[pb], acc.at[dstq[q].at[0]], ssem[pb]
                        ).wait()
                else:
                    pltpu.make_async_copy(
                        rowb[pb], acc.at[dstq[q].at[0]], ssem[pb]
                    ).wait()
                if b + 2 < _GC:
                    nb = (p * _GC + b + 2) % _NBUF
                    pltpu.async_copy(m01_hbm.at[srcq[q].at[b + 2]], rowb[nb], gsem[nb])
                elif nxt is not None:
                    nq, ngrow = nxt
                    if b == _GC - 2:
                        wait_idx(ngrow, nq, isem[nq])
                    nb = (p * _GC + b + 2) % _NBUF
                    pltpu.async_copy(
                        m01_hbm.at[srcq[nq].at[b + 2 - _GC]], rowb[nb], gsem[nb]
                    )
                pltpu.async_copy(rowb[bb], acc.at[dstq[q].at[b]], ssem[bb], add=True)

        pltpu.sync_copy(z_hbm, acc.at[pl.ds(rb, _RPT)])
        pltpu.sync_copy(src_hbm.at[pl.ds(cbase + rowbase, _GC)], srcq[0])
        pltpu.sync_copy(dst_hbm.at[pl.ds(rowbase, _GC)], dstq[0])
        for b in range(_NBUF - 1):
            pltpu.async_copy(m01_hbm.at[srcq[0].at[b]], rowb[b], gsem[b])
        load_idx(rowbase + _GC, 1, isem[1])
        load_idx(rowbase + 2 * _GC, 2, isem[2])
        plsc.subcore_barrier()

        def triple(k, carry):
            gb = rowbase + k * _TRI
            jb = k * _TRI
            process(0, 0, jb, (1, gb + _GC))
            load_idx(gb + 3 * _GC, 0, isem[0])
            process(1, 1, jb + _GC, (2, gb + 2 * _GC))

            @pl.when(k < _NTRI - 1)
            def _():
                load_idx(gb + 4 * _GC, 1, isem[1])

            process(2, 2, jb + 2 * _GC, (0, gb + 3 * _GC))

            @pl.when(k < _NTRI - 1)
            def _():
                load_idx(gb + 5 * _GC, 2, isem[2])

            return carry

        lax.fori_loop(0, _NTRI, triple, 0)
        process(0, 0, (_NG - 1) * _GC, None)
        lastb = (_CH - 1) % _NBUF
        pltpu.make_async_copy(rowb[lastb], acc.at[dstq[0].at[0]], ssem[lastb]).wait()
        plsc.subcore_barrier()

        @pl.when(c == 0)
        def _():
            pltpu.sync_copy(acc.at[pl.ds(rb, _RPT)], out0_hbm.at[pl.ds(rb, _RPT)])

        @pl.when(c == 1)
        def _():
            pltpu.sync_copy(acc.at[pl.ds(rb, _RPT)], out1_hbm.at[pl.ds(rb, _RPT)])

    return scatter_kernel(m01, srcall, dst, zrows)


def _emap(v):
    sp = v[:, 1:]
    n = jnp.sqrt(jnp.clip(jnp.sum(sp * sp, axis=1, keepdims=True), _EPS))
    e = jnp.exp(n)
    ei = 1.0 / e
    x0 = 0.5 * (e + ei)
    rest = 0.5 * (e - ei) * sp / n
    return jnp.concatenate([x0, rest], axis=1)


def _tc3_body(a0_ref, a1_ref, m0_ref, m1_ref, out_ref):
    A0 = a0_ref[...]
    A1 = a1_ref[...]
    M0 = m0_ref[...]
    M1 = m1_ref[...]
    deg = A0[:, _D : _D + 1] + M0[:, _D : _D + 1]
    h1 = _emap((A0[:, :_D] + M0[:, :_D]) / deg)
    h2 = _emap((A1[:, :_D] + M1[:, :_D]) / deg)
    out_ref[...] = jnp.concatenate([h1, h2], axis=1)


def _tc3(A0, A1, M0, M1):
    BM = 1000
    return pl.pallas_call(
        _tc3_body,
        grid=(_N // BM,),
        in_specs=[pl.BlockSpec((BM, _W), lambda i: (i, 0))] * 4,
        out_specs=pl.BlockSpec((BM, 2 * _D), lambda i: (i, 0)),
        out_shape=jax.ShapeDtypeStruct((_N, 2 * _D), jnp.float32),
    )(A0, A1, M0, M1)


def kernel(edge_index, emb_weight, T_weight, ugr_weight, sps, W1, W2):
    x2_items = _tc1(T_weight, sps)
    X2 = jnp.concatenate([ugr_weight, x2_items], axis=0)
    M0, M1 = _tc2(emb_weight, X2, W1, W2)
    m01 = jnp.concatenate([M0, M1], axis=0)
    src = edge_index[0]
    dst = edge_index[1]
    src2d = src.reshape(_E // _K, _K)
    dst2d = dst.reshape(_E // _K, _K)
    srcall = jnp.concatenate([src2d, src2d + _N], axis=0)
    zrows = jnp.zeros((_RPT, _W), jnp.float32)
    A0, A1 = _sc_scatter(m01, srcall, dst2d, zrows)
    return _tc3(A0[:_N], A1[:_N], M0, M1)


# speedup vs baseline: 8.2827x; 1.0638x over previous
import functools

import jax
import jax.numpy as jnp
from jax import lax
from jax.experimental import pallas as pl
from jax.experimental.pallas import tpu as pltpu
from jax.experimental.pallas import tpu_sc as plsc

_N = 10000
_D = 128
_E = 320000
_NT = 1000
_NI = 5000
_W = 144
_EPS = 1e-15

_NS = 16
_K = 80
_CROWS = _E // _K
_CPC = _CROWS // 2
_CH = _CPC // _NS
_ACCR = 10000
_RPT = _ACCR // _NS
_GC = 5
_NG = _CH // _GC


def _tc1_body(t_ref, sps_ref, out_ref):
    T = t_ref[...]
    rest = T[:, 1:]
    x0 = jnp.sqrt(1.0 + jnp.sum(rest * rest, axis=1, keepdims=True))
    y = rest / (x0 + 1.0)
    y2 = jnp.sum(y * y, axis=1, keepdims=True)
    xk = 2.0 * y / (1.0 + y2)
    g = 1.0 / jnp.sqrt(jnp.clip(1.0 - jnp.sum(xk * xk, axis=1, keepdims=True), _EPS))
    G = jnp.concatenate([g * xk, g], axis=1)
    R = jnp.dot(sps_ref[...], G, preferred_element_type=jnp.float32)
    num = R[:, : _D - 1]
    den = jnp.clip(R[:, _D - 1 :], _EPS)
    mean = num / den
    m2 = jnp.sum(mean * mean, axis=1, keepdims=True)
    kp = mean / (1.0 + jnp.sqrt(jnp.clip(1.0 - m2, _EPS)))
    k2 = jnp.sum(kp * kp, axis=1, keepdims=True)
    dn = jnp.clip(1.0 - k2, _EPS)
    out_ref[...] = jnp.concatenate([(1.0 + k2) / dn, 2.0 * kp / dn], axis=1)


def _tc1(T_weight, sps):
    BM = 1000
    return pl.pallas_call(
        _tc1_body,
        grid=(_NI // BM,),
        in_specs=[
            pl.BlockSpec((_NT, _D), lambda i: (0, 0)),
            pl.BlockSpec((BM, _NT), lambda i: (i, 0)),
        ],
        out_specs=pl.BlockSpec((BM, _D), lambda i: (i, 0)),
        out_shape=jax.ShapeDtypeStruct((_NI, _D), jnp.float32),
    )(T_weight, sps)


def _msg(X, Wm):
    rest = X[:, 1:]
    r2 = jnp.sum(rest * rest, axis=1, keepdims=True)
    x0 = jnp.clip(jnp.sqrt(1.0 + r2), 1.0 + 1e-7)
    dist = jnp.log(x0 + jnp.sqrt((x0 - 1.0) * (x0 + 1.0)))
    rn = jnp.sqrt(jnp.clip(r2, _EPS))
    u = dist * rest / rn
    t = jnp.concatenate([jnp.zeros_like(X[:, :1]), u], axis=1)
    return jnp.dot(t, Wm, preferred_element_type=jnp.float32)


def _tc2_body(x_ref, w_ref, m_ref):
    ones = jnp.ones((x_ref.shape[0], _W - _D), jnp.float32)
    m_ref[...] = jnp.concatenate([_msg(x_ref[...], w_ref[...]), ones], axis=1)


def _tc2(X, Wm):
    BM = 1000
    return pl.pallas_call(
        _tc2_body,
        grid=(_N // BM,),
        in_specs=[
            pl.BlockSpec((BM, _D), lambda i: (i, 0)),
            pl.BlockSpec((_D, _D), lambda i: (0, 0)),
        ],
        out_specs=pl.BlockSpec((BM, _W), lambda i: (i, 0)),
        out_shape=jax.ShapeDtypeStruct((_N, _W), jnp.float32),
    )(X, Wm)


_NBUF = 3
_TRI = 3 * _GC
_NTRI = (_NG - 1) // 3


def _sc_scatter(m, src2d, dst2d, zrows):
    mesh = plsc.VectorSubcoreMesh(core_axis_name="c", subcore_axis_name="s")

    @functools.partial(
        pl.kernel,
        out_type=(
            jax.ShapeDtypeStruct((_ACCR, _W), jnp.float32),
            jax.ShapeDtypeStruct((_ACCR, _W), jnp.float32),
        ),
        mesh=mesh,
        compiler_params=pltpu.CompilerParams(use_tc_tiling_on_sc=False),
        scratch_types=[pltpu.VMEM((_GC, _K), jnp.int32)] * 6
        + [pltpu.VMEM((_K, _W), jnp.float32)] * _NBUF
        + [pltpu.VMEM_SHARED((_ACCR, _W), jnp.float32)]
        + [pltpu.SemaphoreType.DMA] * (3 * _NBUF),
    )
    def scatter_kernel(m01_hbm, src_hbm, dst_hbm, z_hbm,
                       out0_hbm, out1_hbm, *rest):
        srcq = list(rest[0:3])
        dstq = list(rest[3:6])
        rowb = list(rest[6 : 6 + _NBUF])
        acc = rest[6 + _NBUF]
        gsem = list(rest[7 + _NBUF : 7 + 2 * _NBUF])
        ssem = list(rest[7 + 2 * _NBUF : 7 + 3 * _NBUF])
        isem = list(rest[7 + 3 * _NBUF :])
        c = lax.axis_index("c")
        s = lax.axis_index("s")
        rb = s * _RPT
        rowbase = c * _CPC + s * _CH

        def load_idx(grow, q, sem):
            pltpu.async_copy(src_hbm.at[pl.ds(grow, _GC)], srcq[q], sem)
            pltpu.async_copy(dst_hbm.at[pl.ds(grow, _GC)], dstq[q], sem)

        def wait_idx(grow, q, sem):
            pltpu.make_async_copy(src_hbm.at[pl.ds(grow, _GC)], srcq[q], sem).wait()
            pltpu.make_async_copy(dst_hbm.at[pl.ds(grow, _GC)], dstq[q], sem).wait()

        def process(p, q, jbase, nxt):
            for b in range(_GC):
                bb = (p * _GC + b) % _NBUF
                pltpu.make_async_copy(
                    m01_hbm.at[srcq[q].at[b]], rowb[bb], gsem[bb]
                ).wait()
                pb = (p * _GC + b - 1) % _NBUF
                if p == 0 and b == 0:
                    @pl.when(jbase > 0)
                    def _():
                        pltpu.make_async_copy(
                            rowb[pb], acc.at[dstq[q].at[0]], ssem[pb]
                        ).wait()
                else:
                    pltpu.make_async_copy(
                        rowb[pb], acc.at[dstq[q].at[0]], ssem[pb]
                    ).wait()
                if b + 2 < _GC:
                    nb = (p * _GC + b + 2) % _NBUF
                    pltpu.async_copy(m01_hbm.at[srcq[q].at[b + 2]], rowb[nb], gsem[nb])
                elif nxt is not None:
                    nq, ngrow = nxt
                    if b == _GC - 2:
                        wait_idx(ngrow, nq, isem[nq])
                    nb = (p * _GC + b + 2) % _NBUF
                    pltpu.async_copy(
                        m01_hbm.at[srcq[nq].at[b + 2 - _GC]], rowb[nb], gsem[nb]
                    )
                pltpu.async_copy(rowb[bb], acc.at[dstq[q].at[b]], ssem[bb], add=True)

        pltpu.sync_copy(z_hbm, acc.at[pl.ds(rb, _RPT)])
        pltpu.sync_copy(src_hbm.at[pl.ds(rowbase, _GC)], srcq[0])
        pltpu.sync_copy(dst_hbm.at[pl.ds(rowbase, _GC)], dstq[0])
        for b in range(_NBUF - 1):
            pltpu.async_copy(m01_hbm.at[srcq[0].at[b]], rowb[b], gsem[b])
        load_idx(rowbase + _GC, 1, isem[1])
        load_idx(rowbase + 2 * _GC, 2, isem[2])
        plsc.subcore_barrier()

        def triple(k, carry):
            gb = rowbase + k * _TRI
            jb = k * _TRI
            process(0, 0, jb, (1, gb + _GC))
            load_idx(gb + 3 * _GC, 0, isem[0])
            process(1, 1, jb + _GC, (2, gb + 2 * _GC))

            @pl.when(k < _NTRI - 1)
            def _():
                load_idx(gb + 4 * _GC, 1, isem[1])

            process(2, 2, jb + 2 * _GC, (0, gb + 3 * _GC))

            @pl.when(k < _NTRI - 1)
            def _():
                load_idx(gb + 5 * _GC, 2, isem[2])

            return carry

        lax.fori_loop(0, _NTRI, triple, 0)
        process(0, 0, (_NG - 1) * _GC, None)
        lastb = (_CH - 1) % _NBUF
        pltpu.make_async_copy(rowb[lastb], acc.at[dstq[0].at[0]], ssem[lastb]).wait()
        plsc.subcore_barrier()

        @pl.when(c == 0)
        def _():
            pltpu.sync_copy(acc.at[pl.ds(rb, _RPT)], out0_hbm.at[pl.ds(rb, _RPT)])

        @pl.when(c == 1)
        def _():
            pltpu.sync_copy(acc.at[pl.ds(rb, _RPT)], out1_hbm.at[pl.ds(rb, _RPT)])

    return scatter_kernel(m, src2d, dst2d, zrows)


def _emap(v):
    sp = v[:, 1:]
    n = jnp.sqrt(jnp.clip(jnp.sum(sp * sp, axis=1, keepdims=True), _EPS))
    e = jnp.exp(n)
    ei = 1.0 / e
    x0 = 0.5 * (e + ei)
    rest = 0.5 * (e - ei) * sp / n
    return jnp.concatenate([x0, rest], axis=1)


def _tc3_body(a0a_ref, a0b_ref, a1a_ref, a1b_ref, m0_ref, m1_ref, out_ref):
    A0 = a0a_ref[...] + a0b_ref[...]
    A1 = a1a_ref[...] + a1b_ref[...]
    M0 = m0_ref[...]
    M1 = m1_ref[...]
    deg = A0[:, _D : _D + 1] + M0[:, _D : _D + 1]
    h1 = _emap((A0[:, :_D] + M0[:, :_D]) / deg)
    h2 = _emap((A1[:, :_D] + M1[:, :_D]) / deg)
    out_ref[...] = jnp.concatenate([h1, h2], axis=1)


def _tc3(A0a, A0b, A1a, A1b, M0, M1):
    BM = 1000
    return pl.pallas_call(
        _tc3_body,
        grid=(_N // BM,),
        in_specs=[pl.BlockSpec((BM, _W), lambda i: (i, 0))] * 6,
        out_specs=pl.BlockSpec((BM, 2 * _D), lambda i: (i, 0)),
        out_shape=jax.ShapeDtypeStruct((_N, 2 * _D), jnp.float32),
    )(A0a, A0b, A1a, A1b, M0, M1)


def kernel(edge_index, emb_weight, T_weight, ugr_weight, sps, W1, W2):
    src2d = edge_index[0].reshape(_CROWS, _K)
    dst2d = edge_index[1].reshape(_CROWS, _K)
    zrows = jnp.zeros((_RPT, _W), jnp.float32)

    M0 = _tc2(emb_weight, W1)
    A0a, A0b = _sc_scatter(M0, src2d, dst2d, zrows)
    x2_items = _tc1(T_weight, sps)
    X2 = jnp.concatenate([ugr_weight, x2_items], axis=0)
    M1 = _tc2(X2, W2)
    A1a, A1b = _sc_scatter(M1, src2d, dst2d, zrows)
    return _tc3(A0a, A0b, A1a, A1b, M0, M1)
